# Initial kernel scaffold; baseline (speedup 1.0000x reference)
import jax
import jax.numpy as jnp
from jax.experimental import pallas as pl


def kernel(x, batch_ids, W, b):
    raise NotImplementedError("write your pallas kernel here")


# speedup vs baseline: 4.3158x; 4.3158x over previous
import functools

import jax
import jax.numpy as jnp
from jax import lax
from jax.experimental import pallas as pl
from jax.experimental.pallas import tpu as pltpu
from jax.experimental.pallas import tpu_sc as plsc

NUM_NODES = 100000
NUM_GRAPHS_K = 1024
NUM_WORKERS = 16
LANES = 16
CHUNK = 6256
NVEC_MAIN = CHUNK // LANES
LAST_BASE = CHUNK * (NUM_WORKERS - 1)
LAST_N = NUM_NODES - LAST_BASE
NVEC_LAST = LAST_N // LANES
NUM_REDUCERS = 8
BINS_PER_RED = NUM_GRAPHS_K // NUM_REDUCERS


def _body(x_hbm, ids_hbm, wb_hbm, out_hbm,
          idx_v, x_v, acc_v, part_v, out_v, wb_v, shared):
    sid = lax.axis_index("s")

    def zero_body(i, c):
        acc_v[pl.ds(i * LANES, LANES)] = jnp.zeros((LANES,), jnp.float32)
        return c
    lax.fori_loop(0, NUM_GRAPHS_K // LANES, zero_body, 0)

    def process(base, nvec):
        n = nvec * LANES
        pltpu.sync_copy(ids_hbm.at[pl.ds(base, n)], idx_v.at[pl.ds(0, n)])
        pltpu.sync_copy(x_hbm.at[pl.ds(base, n)], x_v.at[pl.ds(0, n)])

        def scat_body(i, c):
            idx = idx_v[pl.ds(i * LANES, LANES)]
            xv = x_v[pl.ds(i * LANES, LANES)]
            plsc.addupdate_scatter(acc_v, [idx], xv)
            return c
        lax.fori_loop(0, nvec, scat_body, 0)

    @pl.when(sid < NUM_WORKERS - 1)
    def _():
        process(sid * CHUNK, NVEC_MAIN)

    @pl.when(sid == NUM_WORKERS - 1)
    def _():
        process(LAST_BASE, NVEC_LAST)

    pltpu.sync_copy(acc_v, shared.at[pl.ds(sid * NUM_GRAPHS_K, NUM_GRAPHS_K)])
    plsc.subcore_barrier()

    @pl.when(sid < NUM_REDUCERS)
    def _():
        bin_base = sid * BINS_PER_RED
        for r in range(NUM_WORKERS):
            pltpu.sync_copy(
                shared.at[pl.ds(r * NUM_GRAPHS_K + bin_base, BINS_PER_RED)],
                part_v.at[pl.ds(r * BINS_PER_RED, BINS_PER_RED)])
        pltpu.sync_copy(wb_hbm, wb_v)
        wv = wb_v[pl.ds(0, LANES)]
        bv = wb_v[pl.ds(LANES, LANES)]

        for j in range(BINS_PER_RED // LANES):
            def red_body(r, s):
                return s + part_v[pl.ds(r * BINS_PER_RED + j * LANES, LANES)]
            s = lax.fori_loop(0, NUM_WORKERS, red_body,
                              jnp.zeros((LANES,), jnp.float32))
            out_v[pl.ds(j * LANES, LANES)] = s * wv + bv

        pltpu.sync_copy(out_v, out_hbm.at[pl.ds(bin_base, BINS_PER_RED)])


@jax.jit
def _run(xf, ids, wb):
    mesh = plsc.VectorSubcoreMesh(core_axis_name="c", subcore_axis_name="s",
                                  num_cores=1)
    f = pl.kernel(
        _body,
        out_type=jax.ShapeDtypeStruct((NUM_GRAPHS_K,), jnp.float32),
        mesh=mesh,
        compiler_params=pltpu.CompilerParams(needs_layout_passes=False),
        scratch_types=[
            pltpu.VMEM((CHUNK,), jnp.int32),
            pltpu.VMEM((CHUNK,), jnp.float32),
            pltpu.VMEM((NUM_GRAPHS_K,), jnp.float32),
            pltpu.VMEM((NUM_WORKERS * BINS_PER_RED,), jnp.float32),
            pltpu.VMEM((BINS_PER_RED,), jnp.float32),
            pltpu.VMEM((2 * LANES,), jnp.float32),
            pltpu.VMEM_SHARED((NUM_WORKERS * NUM_GRAPHS_K,), jnp.float32),
        ],
    )
    return f(xf, ids, wb)


def kernel(x, batch_ids, W, b):
    xf = x.reshape(NUM_NODES)
    ids = batch_ids.astype(jnp.int32)
    wb = jnp.concatenate([
        jnp.broadcast_to(W.reshape(1), (LANES,)),
        jnp.broadcast_to(b.reshape(1), (LANES,)),
    ])
    out = _run(xf, ids, wb)
    return out.reshape(NUM_GRAPHS_K, 1)
